# baseline (speedup 1.0000x reference)
import functools
import jax
import jax.numpy as jnp
from jax import lax
from jax.experimental import pallas as pl
from jax.experimental.pallas import tpu as pltpu
from jax.experimental.pallas import tpu_sc as plsc

K = 100
B, H, W = 8, 96, 320
C_IN, HC, NCLS, NREG = 64, 256, 3, 8
HW = H * W
NPT = K * B
NW = 32
NB4 = (H + 2) * (W + 2)
NB8 = (H // 2 + 1) * (W // 2 + 1)
NB16 = (H // 4 + 1) * (W // 4 + 1)


def _cls_head_body(x_ref, w3_ref, alpha_ref, beta_ref, w2_ref, b2_ref, out_ref):
    s = pl.program_id(1)
    r0 = s * 16
    rows = x_ref[0, pl.ds(r0, 18), :, :]
    sh = jnp.concatenate(
        [rows[0:16], rows[1:17], rows[2:18]], axis=2)
    acc = jnp.zeros((16 * W, HC), jnp.float32)
    for dx in range(3):
        blk = sh[:, dx:dx + W, :].reshape(16 * W, 192)
        acc = acc + jnp.dot(blk, w3_ref[dx],
                            preferred_element_type=jnp.float32)
    h = jnp.maximum(acc * alpha_ref[:] + beta_ref[:], 0.0)
    logits = jnp.dot(h, w2_ref[:], preferred_element_type=jnp.float32) \
        + b2_ref[:]
    heat = jnp.clip(jax.nn.sigmoid(logits), 1e-4, 1.0 - 1e-4)
    out_ref[0] = heat.reshape(16, W, NCLS)


def _cls_head(x4p, w3, alpha, beta, w2, b2):
    return pl.pallas_call(
        _cls_head_body,
        grid=(B, H // 16),
        in_specs=[
            pl.BlockSpec((1, H + 2, W + 2, C_IN), lambda b, s: (b, 0, 0, 0)),
            pl.BlockSpec((3, 192, HC), lambda b, s: (0, 0, 0)),
            pl.BlockSpec((1, HC), lambda b, s: (0, 0)),
            pl.BlockSpec((1, HC), lambda b, s: (0, 0)),
            pl.BlockSpec((HC, NCLS), lambda b, s: (0, 0)),
            pl.BlockSpec((1, NCLS), lambda b, s: (0, 0)),
        ],
        out_specs=pl.BlockSpec((1, 16, W, NCLS), lambda b, s: (b, s, 0, 0)),
        out_shape=jax.ShapeDtypeStruct((B, H, W, NCLS), jnp.float32),
    )(x4p, w3, alpha, beta, w2, b2)


def _topk_body(heat_ref, scores_ref, clses_ref, ys_ref, xs_ref,
               i4_ref, i8_ref, i16_ref, w8_ref, w16_ref,
               scr_ref, m1_ref):
    heat = heat_ref[...]
    neg = jnp.full_like(heat, -1.0)

    def shift(a, d, ax):
        lo = [slice(None)] * 4
        hi = [slice(None)] * 4
        lo[ax] = slice(d, None) if d > 0 else slice(0, a.shape[ax] + d)
        hi[ax] = slice(0, abs(d))
        pad = neg[tuple(hi)]
        parts = [a[tuple(lo)], pad] if d > 0 else [pad, a[tuple(lo)]]
        return jnp.concatenate(parts, axis=ax)

    hx = jnp.maximum(heat, jnp.maximum(shift(heat, 1, 3), shift(heat, -1, 3)))
    hmax = jnp.maximum(hx, jnp.maximum(shift(hx, 1, 2), shift(hx, -1, 2)))
    nms = jnp.where(heat >= hmax, heat, 0.0)
    scr_ref[...] = nms
    m1_ref[...] = jnp.max(nms, axis=3).reshape(B, NCLS * H).T

    cyi2 = jax.lax.broadcasted_iota(jnp.int32, (NCLS * H, B), 0)
    bi = jax.lax.broadcasted_iota(jnp.int32, (1, B), 1)
    xi = jax.lax.broadcasted_iota(jnp.int32, (1, W), 1)

    xir = jax.lax.broadcasted_iota(jnp.int32, (B, W), 1)

    def step(k, _):
        m1 = m1_ref[...]
        vmax = jnp.max(m1, axis=0, keepdims=True)
        cysel = jnp.min(jnp.where(m1 >= vmax, cyi2, NCLS * H),
                        axis=0, keepdims=True)
        rows = []
        cys = []
        for b in range(B):
            cyb = cysel[0, b]
            cys.append((cyb // H, cyb % H))
            rows.append(scr_ref[b, cys[b][0], pl.ds(cys[b][1], 1), :])
        rowmat = jnp.concatenate(rows, axis=0)
        vcol = jnp.transpose(vmax, (1, 0))
        xcol = jnp.min(jnp.where(rowmat >= vcol, xir, W), axis=1,
                       keepdims=True)
        newmat = jnp.where(xir == xcol, -1.0, rowmat)
        nmcol = jnp.max(newmat, axis=1, keepdims=True)
        for b in range(B):
            scr_ref[b, cys[b][0], pl.ds(cys[b][1], 1), :] = newmat[b:b + 1]
        scores_ref[pl.ds(k, 1), :] = vmax
        clses_ref[pl.ds(k, 1), :] = (cysel // H).astype(jnp.float32)
        ys_ref[pl.ds(k, 1), :] = (cysel % H).astype(jnp.float32)
        xs_ref[pl.ds(k, 1), :] = jnp.transpose(xcol, (1, 0)).astype(
            jnp.float32)
        m1_ref[...] = jnp.where(cyi2 == cysel,
                                jnp.transpose(nmcol, (1, 0)), m1)
        return 0

    jax.lax.fori_loop(0, K, step, 0)

    bofs = jax.lax.broadcasted_iota(jnp.int32, (K, B), 1)
    yi = ys_ref[...].astype(jnp.int32)
    xi_i = xs_ref[...].astype(jnp.int32)
    for dy in range(3):
        for dx in range(3):
            i4_ref[dy * 3 + dx] = (bofs * NB4
                                   + (yi + dy) * (W + 2) + (xi_i + dx))
    x8 = jnp.minimum(xi_i, W - 2)
    y8 = jnp.minimum(yi, H - 2)
    x80, y80 = x8 // 2, y8 // 2
    fx8 = (x8 % 2).astype(jnp.float32) * 0.5
    fy8 = (y8 % 2).astype(jnp.float32) * 0.5
    x16 = jnp.minimum(xi_i, W - 4)
    y16 = jnp.minimum(yi, H - 4)
    x160, y160 = x16 // 4, y16 // 4
    fx16 = (x16 % 4).astype(jnp.float32) * 0.25
    fy16 = (y16 % 4).astype(jnp.float32) * 0.25
    for dy in range(2):
        for dx in range(2):
            j = dy * 2 + dx
            i8_ref[j] = (bofs * NB8 + (y80 + dy) * (W // 2 + 1)
                         + (x80 + dx))
            w8_ref[j] = ((fy8 if dy else 1.0 - fy8)
                         * (fx8 if dx else 1.0 - fx8))
            i16_ref[j] = (bofs * NB16 + (y160 + dy) * (W // 4 + 1)
                          + (x160 + dx))
            w16_ref[j] = ((fy16 if dy else 1.0 - fy16)
                          * (fx16 if dx else 1.0 - fx16))


def _topk(heat):
    return pl.pallas_call(
        _topk_body,
        out_shape=[jax.ShapeDtypeStruct((K, B), jnp.float32)] * 4 + [
            jax.ShapeDtypeStruct((9, K, B), jnp.int32),
            jax.ShapeDtypeStruct((4, K, B), jnp.int32),
            jax.ShapeDtypeStruct((4, K, B), jnp.int32),
            jax.ShapeDtypeStruct((4, K, B), jnp.float32),
            jax.ShapeDtypeStruct((4, K, B), jnp.float32),
        ],
        scratch_shapes=[
            pltpu.VMEM((B, NCLS, H, W), jnp.float32),
            pltpu.VMEM((NCLS * H, B), jnp.float32),
        ],
    )(heat)


N4, P4C = 225, 240
N8 = 100


def _sc_gather_body(i4, i8, i16, t4, t8, t16, o4, o8, o16,
                    idx4_v, r4_v, idx8_v, r8_v, idx16_v, r16_v, sem):
    wid = lax.axis_index("s") * 2 + lax.axis_index("c")
    pltpu.sync_copy(i4.at[wid], idx4_v)
    for c in range(2):
        pltpu.async_copy(t4.at[idx4_v.at[c]],
                         r4_v.at[pl.ds(120 * c, 120)], sem).wait()
    pltpu.sync_copy(r4_v.at[pl.ds(0, 232)], o4.at[wid])
    pltpu.sync_copy(i8.at[wid], idx8_v)
    pltpu.async_copy(t8.at[idx8_v], r8_v, sem).wait()
    pltpu.sync_copy(r8_v, o8.at[wid])
    pltpu.sync_copy(i16.at[wid], idx16_v)
    pltpu.async_copy(t16.at[idx16_v], r16_v, sem).wait()
    pltpu.sync_copy(r16_v, o16.at[wid])


@functools.lru_cache(maxsize=1)
def _make_sc_gather():
    return pl.kernel(
        _sc_gather_body,
        out_type=[
            jax.ShapeDtypeStruct((NW, 232, C_IN), jnp.float32),
            jax.ShapeDtypeStruct((NW, 104, 128), jnp.float32),
            jax.ShapeDtypeStruct((NW, 104, 256), jnp.float32),
        ],
        mesh=plsc.VectorSubcoreMesh(core_axis_name="c",
                                    subcore_axis_name="s"),
        compiler_params=pltpu.CompilerParams(use_tc_tiling_on_sc=False),
        scratch_types=[
            pltpu.VMEM((2, 120), jnp.int32),
            pltpu.VMEM((P4C, C_IN), jnp.float32),
            pltpu.VMEM((104,), jnp.int32),
            pltpu.VMEM((104, 128), jnp.float32),
            pltpu.VMEM((104,), jnp.int32),
            pltpu.VMEM((104, 256), jnp.float32),
            pltpu.SemaphoreType.DMA,
        ],
    )


def _head_body(p_ref, r8_ref, r16_ref, w8c_ref, w16c_ref, w576_ref,
               alpha_ref, beta_ref, bw4_ref, bw8_ref, bw16_ref, bb_ref,
               out_ref):
    reg = jnp.dot(p_ref[...], w576_ref[...],
                  preferred_element_type=jnp.float32)
    reg = jnp.maximum(reg * alpha_ref[:] + beta_ref[:], 0.0)
    o = jnp.dot(reg, bw4_ref[...], preferred_element_type=jnp.float32) \
        + bb_ref[:]
    z8 = jnp.dot(r8_ref[...], bw8_ref[...],
                 preferred_element_type=jnp.float32) * w8c_ref[...]
    z8 = z8.reshape(NPT, 4, NREG)
    o = o + z8[:, 0] + z8[:, 1] + z8[:, 2] + z8[:, 3]
    z16 = jnp.dot(r16_ref[...], bw16_ref[...],
                  preferred_element_type=jnp.float32) * w16c_ref[...]
    z16 = z16.reshape(NPT, 4, NREG)
    o = o + z16[:, 0] + z16[:, 1] + z16[:, 2] + z16[:, 3]
    li = jax.lax.broadcasted_iota(jnp.int32, (NPT, NREG), 1)
    sig = jax.nn.sigmoid(o) - 0.5
    orimask = li >= 6
    orivals = jnp.where(orimask, o, 0.0)
    nrm = jnp.sqrt(jnp.sum(orivals * orivals, axis=1, keepdims=True))
    orin = o / jnp.maximum(nrm, 1e-12)
    o = jnp.where((li >= 3) & (li < 6), sig, o)
    out_ref[...] = jnp.where(orimask, orin, o)


def _head(p576, r8, r16, w8c, w16c, w576, alpha, beta, bw4, bw8, bw16, bb):
    return pl.pallas_call(
        _head_body,
        out_shape=jax.ShapeDtypeStruct((NPT, NREG), jnp.float32),
    )(p576, r8, r16, w8c, w16c, w576, alpha, beta, bw4, bw8, bw16, bb)


@jax.jit
def kernel(up_level16, up_level8, up_level4, cls_w1, cls_b1, cls_bn_g,
           cls_bn_b, cls_bn_m, cls_bn_v, cls_w2, cls_b2, reg_w1, reg_b1,
           reg_bn_g, reg_bn_b, reg_bn_m, reg_bn_v, box_w, box_b):
    x4 = jnp.transpose(up_level4, (0, 2, 3, 1))
    x4p = jnp.pad(x4, ((0, 0), (1, 1), (1, 1), (0, 0)))
    u8 = jnp.transpose(up_level8, (0, 2, 3, 1))
    u8p = jnp.pad(u8, ((0, 0), (0, 1), (0, 1), (0, 0)), mode='edge')
    u16 = jnp.transpose(up_level16, (0, 2, 3, 1))
    u16p = jnp.pad(u16, ((0, 0), (0, 1), (0, 1), (0, 0)), mode='edge')

    def fold_bn(g, bta, m, v, b1):
        a = g * jax.lax.rsqrt(v + 1e-5)
        return a, (b1 - m) * a + bta

    ca, cb = fold_bn(cls_bn_g, cls_bn_b, cls_bn_m, cls_bn_v, cls_b1)
    ra, rb = fold_bn(reg_bn_g, reg_bn_b, reg_bn_m, reg_bn_v, reg_b1)
    w1t = jnp.transpose(cls_w1, (2, 3, 1, 0))
    w3 = jnp.transpose(w1t, (1, 0, 2, 3)).reshape(3, 192, HC)
    w2 = jnp.transpose(cls_w2[:, :, 0, 0], (1, 0))
    w576 = jnp.transpose(reg_w1, (2, 3, 1, 0)).reshape(576, HC)
    bw = jnp.transpose(box_w[:, :, 0, 0], (1, 0))

    heat = _cls_head(x4p, w3, ca.reshape(1, HC), cb.reshape(1, HC),
                     w2, cls_b2.reshape(1, NCLS))
    heat = jnp.transpose(heat, (0, 3, 1, 2))
    (scores, clses, ysk, xsk, i4, i8, i16, w8, w16) = _topk(heat)

    I4 = jnp.transpose(i4, (1, 2, 0)).reshape(NW, 225)
    I4 = jnp.pad(I4, ((0, 0), (0, 15))).reshape(NW, 2, 120)
    I8 = jnp.pad(jnp.transpose(i8, (1, 2, 0)).reshape(NW, 100),
                 ((0, 0), (0, 4)))
    I16 = jnp.pad(jnp.transpose(i16, (1, 2, 0)).reshape(NW, 100),
                  ((0, 0), (0, 4)))
    t4 = x4p.reshape(B * NB4, C_IN)
    t8 = u8p.reshape(B * NB8, 128)
    t16 = u16p.reshape(B * NB16, 256)
    o4, o8, o16 = _make_sc_gather()(I4, I8, I16, t4, t8, t16)

    w8c = jnp.transpose(w8, (1, 2, 0)).reshape(4 * NPT, 1)
    w16c = jnp.transpose(w16, (1, 2, 0)).reshape(4 * NPT, 1)
    head = _head(o4[:, :225].reshape(NPT, 576),
                 o8[:, :100].reshape(4 * NPT, 128),
                 o16[:, :100].reshape(4 * NPT, 256), w8c, w16c, w576,
                 ra.reshape(1, HC), rb.reshape(1, HC),
                 bw[0:HC], bw[HC:HC + 128], bw[HC + 128:],
                 box_b.reshape(1, NREG))
    head_reg = jnp.transpose(head.reshape(K, B, NREG), (1, 2, 0))
    return (head_reg, scores.T, clses.T, ysk.T, xsk.T)


# speedup vs baseline: 1.0150x; 1.0150x over previous
import functools
import jax
import jax.numpy as jnp
from jax import lax
from jax.experimental import pallas as pl
from jax.experimental.pallas import tpu as pltpu
from jax.experimental.pallas import tpu_sc as plsc

K = 100
B, H, W = 8, 96, 320
C_IN, HC, NCLS, NREG = 64, 256, 3, 8
HW = H * W
NPT = K * B
NW = 32
NB4 = (H + 2) * (W + 2)
NB8 = (H // 2 + 1) * (W // 2 + 1)
NB16 = (H // 4 + 1) * (W // 4 + 1)


def _cls_head_body(x_ref, w3_ref, alpha_ref, beta_ref, w2_ref, b2_ref, out_ref):
    s = pl.program_id(1)
    r0 = s * 16
    rows = x_ref[0, pl.ds(r0, 18), :, :]
    sh = jnp.concatenate(
        [rows[0:16], rows[1:17], rows[2:18]], axis=2)
    acc = jnp.zeros((16 * W, HC), jnp.float32)
    for dx in range(3):
        blk = sh[:, dx:dx + W, :].reshape(16 * W, 192)
        acc = acc + jnp.dot(blk, w3_ref[dx],
                            preferred_element_type=jnp.float32)
    h = jnp.maximum(acc * alpha_ref[:] + beta_ref[:], 0.0)
    logits = jnp.dot(h, w2_ref[:], preferred_element_type=jnp.float32) \
        + b2_ref[:]
    heat = jnp.clip(jax.nn.sigmoid(logits), 1e-4, 1.0 - 1e-4)
    out_ref[0] = heat.reshape(16, W, NCLS)


def _cls_head(x4p, w3, alpha, beta, w2, b2):
    return pl.pallas_call(
        _cls_head_body,
        grid=(B, H // 16),
        in_specs=[
            pl.BlockSpec((1, H + 2, W + 2, C_IN), lambda b, s: (b, 0, 0, 0)),
            pl.BlockSpec((3, 192, HC), lambda b, s: (0, 0, 0)),
            pl.BlockSpec((1, HC), lambda b, s: (0, 0)),
            pl.BlockSpec((1, HC), lambda b, s: (0, 0)),
            pl.BlockSpec((HC, NCLS), lambda b, s: (0, 0)),
            pl.BlockSpec((1, NCLS), lambda b, s: (0, 0)),
        ],
        out_specs=pl.BlockSpec((1, 16, W, NCLS), lambda b, s: (b, s, 0, 0)),
        out_shape=jax.ShapeDtypeStruct((B, H, W, NCLS), jnp.float32),
    )(x4p, w3, alpha, beta, w2, b2)


def _topk_body(heat_ref, scores_ref, clses_ref, ys_ref, xs_ref,
               i4_ref, i8_ref, i16_ref, w8_ref, w16_ref,
               scr_ref, m1_ref):
    heat = heat_ref[...]
    neg = jnp.full_like(heat, -1.0)

    def shift(a, d, ax):
        lo = [slice(None)] * 4
        hi = [slice(None)] * 4
        lo[ax] = slice(d, None) if d > 0 else slice(0, a.shape[ax] + d)
        hi[ax] = slice(0, abs(d))
        pad = neg[tuple(hi)]
        parts = [a[tuple(lo)], pad] if d > 0 else [pad, a[tuple(lo)]]
        return jnp.concatenate(parts, axis=ax)

    hx = jnp.maximum(heat, jnp.maximum(shift(heat, 1, 3), shift(heat, -1, 3)))
    hmax = jnp.maximum(hx, jnp.maximum(shift(hx, 1, 2), shift(hx, -1, 2)))
    nms = jnp.where(heat >= hmax, heat, 0.0)
    scr_ref[...] = nms
    m1_ref[...] = jnp.max(nms, axis=3).reshape(B, NCLS * H).T

    cyi2 = jax.lax.broadcasted_iota(jnp.int32, (NCLS * H, B), 0)
    bi = jax.lax.broadcasted_iota(jnp.int32, (1, B), 1)
    xi = jax.lax.broadcasted_iota(jnp.int32, (1, W), 1)

    xir = jax.lax.broadcasted_iota(jnp.int32, (B, W), 1)

    def step(k, _):
        m1 = m1_ref[...]
        vmax = jnp.max(m1, axis=0, keepdims=True)
        cysel = jnp.min(jnp.where(m1 >= vmax, cyi2, NCLS * H),
                        axis=0, keepdims=True)
        rows = []
        cys = []
        for b in range(B):
            cyb = cysel[0, b]
            cys.append((cyb // H, cyb % H))
            rows.append(scr_ref[b, cys[b][0], pl.ds(cys[b][1], 1), :])
        rowmat = jnp.concatenate(rows, axis=0)
        vcol = jnp.transpose(vmax, (1, 0))
        xcol = jnp.min(jnp.where(rowmat >= vcol, xir, W), axis=1,
                       keepdims=True)
        newmat = jnp.where(xir == xcol, -1.0, rowmat)
        nmcol = jnp.max(newmat, axis=1, keepdims=True)
        for b in range(B):
            scr_ref[b, cys[b][0], pl.ds(cys[b][1], 1), :] = newmat[b:b + 1]
        scores_ref[pl.ds(k, 1), :] = vmax
        clses_ref[pl.ds(k, 1), :] = (cysel // H).astype(jnp.float32)
        ys_ref[pl.ds(k, 1), :] = (cysel % H).astype(jnp.float32)
        xs_ref[pl.ds(k, 1), :] = jnp.transpose(xcol, (1, 0)).astype(
            jnp.float32)
        m1_ref[...] = jnp.where(cyi2 == cysel,
                                jnp.transpose(nmcol, (1, 0)), m1)
        return 0

    jax.lax.fori_loop(0, K, step, 0)

    bofs = jax.lax.broadcasted_iota(jnp.int32, (K, B), 1)
    yi = ys_ref[...].astype(jnp.int32)
    xi_i = xs_ref[...].astype(jnp.int32)
    for dy in range(3):
        for dx in range(3):
            i4_ref[dy * 3 + dx] = (bofs * NB4
                                   + (yi + dy) * (W + 2) + (xi_i + dx))
    x8 = jnp.minimum(xi_i, W - 2)
    y8 = jnp.minimum(yi, H - 2)
    x80, y80 = x8 // 2, y8 // 2
    fx8 = (x8 % 2).astype(jnp.float32) * 0.5
    fy8 = (y8 % 2).astype(jnp.float32) * 0.5
    x16 = jnp.minimum(xi_i, W - 4)
    y16 = jnp.minimum(yi, H - 4)
    x160, y160 = x16 // 4, y16 // 4
    fx16 = (x16 % 4).astype(jnp.float32) * 0.25
    fy16 = (y16 % 4).astype(jnp.float32) * 0.25
    for dy in range(2):
        for dx in range(2):
            j = dy * 2 + dx
            i8_ref[j] = (bofs * NB8 + (y80 + dy) * (W // 2 + 1)
                         + (x80 + dx))
            w8_ref[j] = ((fy8 if dy else 1.0 - fy8)
                         * (fx8 if dx else 1.0 - fx8))
            i16_ref[j] = (bofs * NB16 + (y160 + dy) * (W // 4 + 1)
                          + (x160 + dx))
            w16_ref[j] = ((fy16 if dy else 1.0 - fy16)
                          * (fx16 if dx else 1.0 - fx16))


def _topk(heat):
    return pl.pallas_call(
        _topk_body,
        out_shape=[jax.ShapeDtypeStruct((K, B), jnp.float32)] * 4 + [
            jax.ShapeDtypeStruct((9, K, B), jnp.int32),
            jax.ShapeDtypeStruct((4, K, B), jnp.int32),
            jax.ShapeDtypeStruct((4, K, B), jnp.int32),
            jax.ShapeDtypeStruct((4, K, B), jnp.float32),
            jax.ShapeDtypeStruct((4, K, B), jnp.float32),
        ],
        scratch_shapes=[
            pltpu.VMEM((B, NCLS, H, W), jnp.float32),
            pltpu.VMEM((NCLS * H, B), jnp.float32),
        ],
    )(heat)


N4, P4C = 225, 240
N8 = 100


def _sc_gather_body(i4, i8, i16, t4, t8, t16, o4, o8, o16,
                    idx4_v, r4_v, idx8_v, r8_v, idx16_v, r16_v, sem):
    wid = lax.axis_index("s") * 2 + lax.axis_index("c")
    pltpu.sync_copy(i4.at[wid], idx4_v)
    pltpu.sync_copy(i8.at[wid], idx8_v)
    pltpu.sync_copy(i16.at[wid], idx16_v)
    handles = []
    for c in range(2):
        handles.append(pltpu.async_copy(
            t4.at[idx4_v.at[c]], r4_v.at[pl.ds(120 * c, 120)], sem))
    handles.append(pltpu.async_copy(t8.at[idx8_v], r8_v, sem))
    handles.append(pltpu.async_copy(t16.at[idx16_v], r16_v, sem))
    for h in handles:
        h.wait()
    pltpu.sync_copy(r4_v.at[pl.ds(0, 232)], o4.at[wid])
    pltpu.sync_copy(r8_v, o8.at[wid])
    pltpu.sync_copy(r16_v, o16.at[wid])


@functools.lru_cache(maxsize=1)
def _make_sc_gather():
    return pl.kernel(
        _sc_gather_body,
        out_type=[
            jax.ShapeDtypeStruct((NW, 232, C_IN), jnp.float32),
            jax.ShapeDtypeStruct((NW, 104, 128), jnp.float32),
            jax.ShapeDtypeStruct((NW, 104, 256), jnp.float32),
        ],
        mesh=plsc.VectorSubcoreMesh(core_axis_name="c",
                                    subcore_axis_name="s"),
        compiler_params=pltpu.CompilerParams(use_tc_tiling_on_sc=False),
        scratch_types=[
            pltpu.VMEM((2, 120), jnp.int32),
            pltpu.VMEM((P4C, C_IN), jnp.float32),
            pltpu.VMEM((104,), jnp.int32),
            pltpu.VMEM((104, 128), jnp.float32),
            pltpu.VMEM((104,), jnp.int32),
            pltpu.VMEM((104, 256), jnp.float32),
            pltpu.SemaphoreType.DMA,
        ],
    )


def _head_body(p_ref, r8_ref, r16_ref, w8c_ref, w16c_ref, w576_ref,
               alpha_ref, beta_ref, bw4_ref, bw8_ref, bw16_ref, bb_ref,
               out_ref):
    reg = jnp.zeros((NPT, HC), jnp.float32)
    for j in range(9):
        reg = reg + jnp.dot(p_ref[j], w576_ref[j],
                            preferred_element_type=jnp.float32)
    reg = jnp.maximum(reg * alpha_ref[:] + beta_ref[:], 0.0)
    o = jnp.dot(reg, bw4_ref[...], preferred_element_type=jnp.float32) \
        + bb_ref[:]
    z8 = jnp.dot(r8_ref[...], bw8_ref[...],
                 preferred_element_type=jnp.float32) * w8c_ref[...]
    z8 = z8.reshape(4, NPT, NREG)
    o = o + z8[0] + z8[1] + z8[2] + z8[3]
    z16 = jnp.dot(r16_ref[...], bw16_ref[...],
                  preferred_element_type=jnp.float32) * w16c_ref[...]
    z16 = z16.reshape(4, NPT, NREG)
    o = o + z16[0] + z16[1] + z16[2] + z16[3]
    li = jax.lax.broadcasted_iota(jnp.int32, (NPT, NREG), 1)
    sig = jax.nn.sigmoid(o) - 0.5
    orimask = li >= 6
    orivals = jnp.where(orimask, o, 0.0)
    nrm = jnp.sqrt(jnp.sum(orivals * orivals, axis=1, keepdims=True))
    orin = o / jnp.maximum(nrm, 1e-12)
    o = jnp.where((li >= 3) & (li < 6), sig, o)
    out_ref[...] = jnp.where(orimask, orin, o)


def _head(p576, r8, r16, w8c, w16c, w576, alpha, beta, bw4, bw8, bw16, bb):
    return pl.pallas_call(
        _head_body,
        out_shape=jax.ShapeDtypeStruct((NPT, NREG), jnp.float32),
    )(p576, r8, r16, w8c, w16c, w576, alpha, beta, bw4, bw8, bw16, bb)


@jax.jit
def kernel(up_level16, up_level8, up_level4, cls_w1, cls_b1, cls_bn_g,
           cls_bn_b, cls_bn_m, cls_bn_v, cls_w2, cls_b2, reg_w1, reg_b1,
           reg_bn_g, reg_bn_b, reg_bn_m, reg_bn_v, box_w, box_b):
    x4 = jnp.transpose(up_level4, (0, 2, 3, 1))
    x4p = jnp.pad(x4, ((0, 0), (1, 1), (1, 1), (0, 0)))
    u8 = jnp.transpose(up_level8, (0, 2, 3, 1))
    u8p = jnp.pad(u8, ((0, 0), (0, 1), (0, 1), (0, 0)), mode='edge')
    u16 = jnp.transpose(up_level16, (0, 2, 3, 1))
    u16p = jnp.pad(u16, ((0, 0), (0, 1), (0, 1), (0, 0)), mode='edge')

    def fold_bn(g, bta, m, v, b1):
        a = g * jax.lax.rsqrt(v + 1e-5)
        return a, (b1 - m) * a + bta

    ca, cb = fold_bn(cls_bn_g, cls_bn_b, cls_bn_m, cls_bn_v, cls_b1)
    ra, rb = fold_bn(reg_bn_g, reg_bn_b, reg_bn_m, reg_bn_v, reg_b1)
    w1t = jnp.transpose(cls_w1, (2, 3, 1, 0))
    w3 = jnp.transpose(w1t, (1, 0, 2, 3)).reshape(3, 192, HC)
    w2 = jnp.transpose(cls_w2[:, :, 0, 0], (1, 0))
    w576 = jnp.transpose(reg_w1, (2, 3, 1, 0)).reshape(9, C_IN, HC)
    bw = jnp.transpose(box_w[:, :, 0, 0], (1, 0))

    heat = _cls_head(x4p, w3, ca.reshape(1, HC), cb.reshape(1, HC),
                     w2, cls_b2.reshape(1, NCLS))
    heat = jnp.transpose(heat, (0, 3, 1, 2))
    (scores, clses, ysk, xsk, i4, i8, i16, w8, w16) = _topk(heat)

    I4 = i4.reshape(NW, 225)
    I4 = jnp.pad(I4, ((0, 0), (0, 15))).reshape(NW, 2, 120)
    I8 = jnp.pad(i8.reshape(NW, 100), ((0, 0), (0, 4)))
    I16 = jnp.pad(i16.reshape(NW, 100), ((0, 0), (0, 4)))
    t4 = x4p.reshape(B * NB4, C_IN)
    t8 = u8p.reshape(B * NB8, 128)
    t16 = u16p.reshape(B * NB16, 256)
    o4, o8, o16 = _make_sc_gather()(I4, I8, I16, t4, t8, t16)

    w8c = w8.reshape(4 * NPT, 1)
    w16c = w16.reshape(4 * NPT, 1)
    head = _head(o4[:, :225].reshape(9, NPT, C_IN),
                 o8[:, :100].reshape(4 * NPT, 128),
                 o16[:, :100].reshape(4 * NPT, 256), w8c, w16c, w576,
                 ra.reshape(1, HC), rb.reshape(1, HC),
                 bw[0:HC], bw[HC:HC + 128], bw[HC + 128:],
                 box_b.reshape(1, NREG))
    head_reg = jnp.transpose(head.reshape(K, B, NREG), (1, 2, 0))
    return (head_reg, scores.T, clses.T, ysk.T, xsk.T)
